# baseline (speedup 1.0000x reference)
import functools

import jax
import jax.numpy as jnp
from jax.experimental import pallas as pl
from jax.experimental.pallas import tpu as pltpu


def _fused_se_kernel(x_ref, w1t_ref, b1r_ref, w2t_ref, b2r_ref, o_ref, *,
                     inv_hw):
    x = x_ref[0]
    pooled = jnp.sum(jnp.sum(x.astype(jnp.float32), axis=-1), axis=-1)
    pooled = (pooled * inv_hw)[None, :]

    y1 = jnp.dot(pooled, w1t_ref[...],
                 preferred_element_type=jnp.float32) + b1r_ref[...]
    y1 = jnp.maximum(y1, 0.0)
    y2 = jnp.dot(y1, w2t_ref[...],
                 preferred_element_type=jnp.float32) + b2r_ref[...]
    gate = jax.nn.sigmoid(y2).astype(o_ref.dtype)

    o_ref[...] = x_ref[...] * gate[0][None, :, None, None]


@jax.jit
def _ca_fused(x, w1, b1, w2, b2):
    B, C, H, W = x.shape
    mid = w1.shape[0]

    w1t = jnp.transpose(w1)
    w2t = jnp.transpose(w2)
    b1r = b1.reshape(1, mid)
    b2r = b2.reshape(1, C)
    inv_hw = 1.0 / float(H * W)

    return pl.pallas_call(
        functools.partial(_fused_se_kernel, inv_hw=inv_hw),
        out_shape=jax.ShapeDtypeStruct((B, C, H, W), x.dtype),
        grid=(B,),
        in_specs=[
            pl.BlockSpec((1, C, H, W), lambda b: (b, 0, 0, 0)),
            pl.BlockSpec((C, mid), lambda b: (0, 0)),
            pl.BlockSpec((1, mid), lambda b: (0, 0)),
            pl.BlockSpec((mid, C), lambda b: (0, 0)),
            pl.BlockSpec((1, C), lambda b: (0, 0)),
        ],
        out_specs=pl.BlockSpec((1, C, H, W), lambda b: (b, 0, 0, 0)),
        compiler_params=pltpu.CompilerParams(
            dimension_semantics=("parallel",)),
    )(x, w1t, b1r, w2t, b2r)


def kernel(x, w1, b1, w2, b2):
    return _ca_fused(x, w1, b1, w2, b2)


# speedup vs baseline: 1.3178x; 1.3178x over previous
import functools

import jax
import jax.numpy as jnp
from jax.experimental import pallas as pl
from jax.experimental.pallas import tpu as pltpu


def _fused_se_kernel(x_ref, w1t_ref, b1r_ref, w2t_ref, b2r_ref, o_ref, *,
                     C, H, W, inv_hw):
    x = x_ref[0].reshape(C, H, W)
    pooled = jnp.sum(jnp.sum(x.astype(jnp.float32), axis=-1), axis=-1)
    pooled = (pooled * inv_hw)[None, :]

    y1 = jnp.dot(pooled, w1t_ref[...],
                 preferred_element_type=jnp.float32) + b1r_ref[...]
    y1 = jnp.maximum(y1, 0.0)
    y2 = jnp.dot(y1, w2t_ref[...],
                 preferred_element_type=jnp.float32) + b2r_ref[...]
    gate = jax.nn.sigmoid(y2).astype(o_ref.dtype)

    o_ref[0] = (x * gate[0][:, None, None]).reshape(C * H, W)


@jax.jit
def _ca_fused(x, w1, b1, w2, b2):
    B, C, H, W = x.shape
    mid = w1.shape[0]
    x3 = x.reshape(B, C * H, W)

    w1t = jnp.transpose(w1)
    w2t = jnp.transpose(w2)
    b1r = b1.reshape(1, mid)
    b2r = b2.reshape(1, C)
    inv_hw = 1.0 / float(H * W)

    out = pl.pallas_call(
        functools.partial(_fused_se_kernel, C=C, H=H, W=W, inv_hw=inv_hw),
        out_shape=jax.ShapeDtypeStruct((B, C * H, W), x.dtype),
        grid=(B,),
        in_specs=[
            pl.BlockSpec((1, C * H, W), lambda b: (b, 0, 0)),
            pl.BlockSpec((C, mid), lambda b: (0, 0)),
            pl.BlockSpec((1, mid), lambda b: (0, 0)),
            pl.BlockSpec((mid, C), lambda b: (0, 0)),
            pl.BlockSpec((1, C), lambda b: (0, 0)),
        ],
        out_specs=pl.BlockSpec((1, C * H, W), lambda b: (b, 0, 0)),
        compiler_params=pltpu.CompilerParams(
            dimension_semantics=("parallel",)),
    )(x3, w1t, b1r, w2t, b2r)

    return out.reshape(B, C, H, W)


def kernel(x, w1, b1, w2, b2):
    return _ca_fused(x, w1, b1, w2, b2)


# speedup vs baseline: 6.2236x; 4.7228x over previous
import functools

import jax
import jax.numpy as jnp
from jax.experimental import pallas as pl
from jax.experimental.pallas import tpu as pltpu


def _fused_se_kernel(x_ref, w1t_ref, b1r_ref, w2t_ref, b2r_ref, o_ref, *,
                     inv_hw):
    H, W, C = x_ref.shape[1:]
    x = x_ref[0].reshape(H * W, C)
    pooled = (jnp.sum(x.astype(jnp.float32), axis=0) * inv_hw)[None, :]

    y1 = jnp.dot(pooled, w1t_ref[...],
                 preferred_element_type=jnp.float32) + b1r_ref[...]
    y1 = jnp.maximum(y1, 0.0)
    y2 = jnp.dot(y1, w2t_ref[...],
                 preferred_element_type=jnp.float32) + b2r_ref[...]
    gate = jax.nn.sigmoid(y2).astype(o_ref.dtype)

    o_ref[...] = x_ref[...] * gate[0][None, None, None, :]


@jax.jit
def _ca_fused(x, w1, b1, w2, b2):
    B, C, H, W = x.shape
    mid = w1.shape[0]
    xt = jnp.transpose(x, (0, 2, 3, 1))

    w1t = jnp.transpose(w1)
    w2t = jnp.transpose(w2)
    b1r = b1.reshape(1, mid)
    b2r = b2.reshape(1, C)
    inv_hw = 1.0 / float(H * W)

    out = pl.pallas_call(
        functools.partial(_fused_se_kernel, inv_hw=inv_hw),
        out_shape=jax.ShapeDtypeStruct((B, H, W, C), x.dtype),
        grid=(B,),
        in_specs=[
            pl.BlockSpec((1, H, W, C), lambda b: (b, 0, 0, 0)),
            pl.BlockSpec((C, mid), lambda b: (0, 0)),
            pl.BlockSpec((1, mid), lambda b: (0, 0)),
            pl.BlockSpec((mid, C), lambda b: (0, 0)),
            pl.BlockSpec((1, C), lambda b: (0, 0)),
        ],
        out_specs=pl.BlockSpec((1, H, W, C), lambda b: (b, 0, 0, 0)),
        compiler_params=pltpu.CompilerParams(
            dimension_semantics=("parallel",)),
    )(xt, w1t, b1r, w2t, b2r)

    return jnp.transpose(out, (0, 3, 1, 2))


def kernel(x, w1, b1, w2, b2):
    return _ca_fused(x, w1, b1, w2, b2)
